# baseline (speedup 1.0000x reference)
from functools import partial

import jax
import jax.numpy as jnp
from jax import lax
from jax.experimental import pallas as pl
from jax.experimental.pallas import tpu as pltpu

_S = 2

_C00 = (((0,), (0,)), ((), ()))
_C11 = (((1,), (1,)), ((), ()))


def _split(x):
    hi = x.astype(jnp.bfloat16).astype(jnp.float32)
    return hi, x - hi


def _ge_kernel(f_ref, w_ref, a_ref, p_ref, q_hbm, emb_ref,
               adj_bf, base_scr, fcol_scr, pvec, qrow_s, sem_q,
               *, iters, G, E):
    b = pl.program_id(0)
    s = pl.program_id(1)
    CH = G // _S

    def eye_f32():
        r = lax.broadcasted_iota(jnp.int32, (E, E), 0)
        c = lax.broadcasted_iota(jnp.int32, (E, E), 1)
        return (r == c).astype(jnp.float32)

    @pl.when(s == 0)
    def _folds():
        eye = eye_f32()

        def t_row(col):
            hi, lo = _split(col)
            return (lax.dot_general(hi, eye, _C00,
                                    preferred_element_type=jnp.float32) +
                    lax.dot_general(lo, eye, _C00,
                                    preferred_element_type=jnp.float32))

        def dot3(row, mat):
            rhi, rlo = _split(row)
            mhi, mlo = _split(mat)
            return (jnp.dot(rhi, mhi, preferred_element_type=jnp.float32) +
                    jnp.dot(rhi, mlo, preferred_element_type=jnp.float32) +
                    jnp.dot(rlo, mhi, preferred_element_type=jnp.float32))

        def dot3c(col, mat):
            chi, clo = _split(col)
            mhi, mlo = _split(mat)
            cd = (((0,), (1,)), ((), ()))
            return (lax.dot_general(chi, mhi, cd,
                                    preferred_element_type=jnp.float32) +
                    lax.dot_general(chi, mlo, cd,
                                    preferred_element_type=jnp.float32) +
                    lax.dot_general(clo, mhi, cd,
                                    preferred_element_type=jnp.float32))

        f_row = f_ref[pl.ds(b, 1), :]
        for k in range(G // E):
            fcol_scr[k * E:(k + 1) * E, :] = lax.dot_general(
                eye, f_row[:, k * E:(k + 1) * E], _C11,
                preferred_element_type=jnp.float32)

        wsel_col = p_ref[4 * E:5 * E, 0:1]
        wew_col = p_ref[4 * E:5 * E, 1:2]
        pvec[0:1, :] = dot3c(jnp.maximum(wew_col, 0.0),
                             p_ref[0:E, :])
        pvec[1:2, :] = dot3c(jnp.maximum(-wew_col, 0.0),
                             p_ref[0:E, :])
        pvec[2:3, :] = dot3(p_ref[5 * E:5 * E + 1, :],
                            p_ref[2 * E:3 * E, :])
        pvec[3:4, :] = dot3(p_ref[5 * E + 1:5 * E + 2, :],
                            p_ref[3 * E:4 * E, :])
        pvec[4:5, :] = t_row(wsel_col)

    w_c = w_ref[0]
    pos = jnp.sum(jnp.maximum(w_c, 0.0), axis=1, keepdims=True)
    neg = jnp.sum(jnp.maximum(-w_c, 0.0), axis=1, keepdims=True)
    f_col = fcol_scr[pl.ds(s * CH, CH), :]
    base_scr[pl.ds(s * CH, CH), :] = (f_col * pvec[4:5, :] +
                                      pos * pvec[0:1, :] +
                                      neg * pvec[1:2, :])
    adj_bf[pl.ds(s * CH, CH), :] = a_ref[0].astype(jnp.bfloat16)

    @pl.when(s == _S - 1)
    def _tail():
        base = base_scr[...]
        adj = adj_bf[...]
        wnbp = p_ref[E:2 * E, :].astype(jnp.bfloat16)

        def body(_, emb):
            vp = jnp.dot(adj, emb.astype(jnp.bfloat16),
                         preferred_element_type=jnp.float32)
            vp = lax.dot_general(vp.astype(jnp.bfloat16), wnbp, _C11,
                                 preferred_element_type=jnp.float32)
            return jnp.maximum(base + vp, 0.0)

        emb = lax.fori_loop(0, iters - 1, body, jnp.maximum(base, 0.0))
        emb_ref[...] = emb

        sum_g = jnp.sum(emb, axis=0, keepdims=True)
        t_all = jnp.sum(sum_g * pvec[2:3, :], axis=1, keepdims=True)
        ehi, elo = _split(emb)
        vhi, vlo = _split(pvec[3:4, :])
        t_act_row = (lax.dot_general(vhi, ehi, _C11,
                                     preferred_element_type=jnp.float32) +
                     lax.dot_general(vhi, elo, _C11,
                                     preferred_element_type=jnp.float32) +
                     lax.dot_general(vlo, ehi, _C11,
                                     preferred_element_type=jnp.float32))
        qrow_s[...] = t_act_row + t_all
        cp_q = pltpu.make_async_copy(qrow_s, q_hbm.at[pl.ds(b, 1), :], sem_q)
        cp_q.start()
        cp_q.wait()


@partial(jax.jit, static_argnames=("iters",))
def _graph_embedder(features, weights, adjacency, params, iters=5):
    wsel, wew, wnbw, wnbp, wqall, wqact, wreduc = params
    B, G = features.shape
    E = wsel.shape[0]
    assert iters >= 1 and G % _S == 0

    f = features.astype(jnp.float32)
    w = weights.astype(jnp.float32)
    a = adjacency.astype(jnp.float32)
    packed = jnp.concatenate(
        [wnbw, wnbp, wqall, wqact,
         jnp.concatenate([wsel, wew, jnp.zeros((E, E - 2), jnp.float32)],
                         axis=1),
         wreduc.reshape(2, E),
         jnp.zeros((6, E), jnp.float32)], axis=0).astype(jnp.float32)
    CH = G // _S

    kern = partial(_ge_kernel, iters=iters, G=G, E=E)

    q, emb_flat = pl.pallas_call(
        kern,
        out_shape=(jax.ShapeDtypeStruct((B, G), jnp.float32),
                   jax.ShapeDtypeStruct((B * G, E), jnp.float32)),
        grid_spec=pltpu.PrefetchScalarGridSpec(
            num_scalar_prefetch=0,
            grid=(B, _S),
            in_specs=[
                pl.BlockSpec((B, G), lambda b, s: (0, 0)),
                pl.BlockSpec((1, CH, G), lambda b, s: (b, s, 0)),
                pl.BlockSpec((1, CH, G), lambda b, s: (b, s, 0)),
                pl.BlockSpec((5 * E + 8, E), lambda b, s: (0, 0)),
            ],
            out_specs=[
                pl.BlockSpec(memory_space=pltpu.MemorySpace.HBM),
                pl.BlockSpec((G, E), lambda b, s: (b, 0)),
            ],
            scratch_shapes=[
                pltpu.VMEM((G, G), jnp.bfloat16),
                pltpu.VMEM((G, E), jnp.float32),
                pltpu.VMEM((G, 1), jnp.float32),
                pltpu.VMEM((8, E), jnp.float32),
                pltpu.VMEM((1, G), jnp.float32),
                pltpu.SemaphoreType.DMA,
            ],
        ),
        compiler_params=pltpu.CompilerParams(
            dimension_semantics=("parallel", "arbitrary"),
            vmem_limit_bytes=64 * 1024 * 1024),
    )(f, w, a, packed)

    emb = emb_flat.reshape(B, G, E)
    return q, emb


def kernel(features, weights, adjacency, w_selected, w_nbweights_ew,
           w_nbweights, w_nbpriors, w_q_allembed, w_q_action, w_q_reduc):
    params = (w_selected, w_nbweights_ew, w_nbweights, w_nbpriors,
              w_q_allembed, w_q_action, w_q_reduc)
    return _graph_embedder(features, weights, adjacency, params, iters=5)


# speedup vs baseline: 1.3118x; 1.3118x over previous
from functools import partial

import jax
import jax.numpy as jnp
from jax import lax
from jax.experimental import pallas as pl
from jax.experimental.pallas import tpu as pltpu

_S = 2

_C00 = (((0,), (0,)), ((), ()))
_C11 = (((1,), (1,)), ((), ()))


def _split(x):
    hi = x.astype(jnp.bfloat16).astype(jnp.float32)
    return hi, x - hi


def _ge_kernel(f_ref, w_ref, a_ref, wselew_ref, wnbw_ref, wnbp_ref,
               wqall_ref, wqact_ref, wreduc_ref, q_hbm, emb_ref,
               adj_bf, base_scr, fcol_scr, pvec, qrow_s, sem_q,
               *, iters, G, E):
    b = pl.program_id(0)
    s = pl.program_id(1)
    CH = G // _S

    def eye_f32():
        r = lax.broadcasted_iota(jnp.int32, (E, E), 0)
        c = lax.broadcasted_iota(jnp.int32, (E, E), 1)
        return (r == c).astype(jnp.float32)

    @pl.when(s == 0)
    def _folds():
        eye = eye_f32()

        def t_row(col):
            hi, lo = _split(col)
            return (lax.dot_general(hi, eye, _C00,
                                    preferred_element_type=jnp.float32) +
                    lax.dot_general(lo, eye, _C00,
                                    preferred_element_type=jnp.float32))

        def dot3(row, mat):
            rhi, rlo = _split(row)
            mhi, mlo = _split(mat)
            return (jnp.dot(rhi, mhi, preferred_element_type=jnp.float32) +
                    jnp.dot(rhi, mlo, preferred_element_type=jnp.float32) +
                    jnp.dot(rlo, mhi, preferred_element_type=jnp.float32))

        def dot3c(col, mat):
            chi, clo = _split(col)
            mhi, mlo = _split(mat)
            cd = (((0,), (1,)), ((), ()))
            return (lax.dot_general(chi, mhi, cd,
                                    preferred_element_type=jnp.float32) +
                    lax.dot_general(chi, mlo, cd,
                                    preferred_element_type=jnp.float32) +
                    lax.dot_general(clo, mhi, cd,
                                    preferred_element_type=jnp.float32))

        f_row = f_ref[pl.ds(b, 1), :]
        for k in range(G // E):
            fcol_scr[k * E:(k + 1) * E, :] = lax.dot_general(
                eye, f_row[:, k * E:(k + 1) * E], _C11,
                preferred_element_type=jnp.float32)

        wsel_col = wselew_ref[:, 0:1]
        wew_col = wselew_ref[:, 1:2]
        pvec[0:1, :] = dot3c(jnp.maximum(wew_col, 0.0),
                             wnbw_ref[...])
        pvec[1:2, :] = dot3c(jnp.maximum(-wew_col, 0.0),
                             wnbw_ref[...])
        pvec[2:3, :] = dot3(wreduc_ref[:, 0:E], wqall_ref[...])
        pvec[3:4, :] = dot3(wreduc_ref[:, E:2 * E], wqact_ref[...])
        pvec[4:5, :] = t_row(wsel_col)

    w_c = w_ref[0]
    pos = jnp.sum(jnp.maximum(w_c, 0.0), axis=1, keepdims=True)
    neg = jnp.sum(jnp.maximum(-w_c, 0.0), axis=1, keepdims=True)
    f_col = fcol_scr[pl.ds(s * CH, CH), :]
    base_scr[pl.ds(s * CH, CH), :] = (f_col * pvec[4:5, :] +
                                      pos * pvec[0:1, :] +
                                      neg * pvec[1:2, :])
    adj_bf[pl.ds(s * CH, CH), :] = a_ref[0].astype(jnp.bfloat16)

    @pl.when(s == _S - 1)
    def _tail():
        base = base_scr[...]
        adj = adj_bf[...]
        wnbp = wnbp_ref[...].astype(jnp.bfloat16)

        def body(_, emb):
            vp = jnp.dot(adj, emb.astype(jnp.bfloat16),
                         preferred_element_type=jnp.float32)
            vp = lax.dot_general(vp.astype(jnp.bfloat16), wnbp, _C11,
                                 preferred_element_type=jnp.float32)
            return jnp.maximum(base + vp, 0.0)

        emb = lax.fori_loop(0, iters - 1, body, jnp.maximum(base, 0.0))
        emb_ref[...] = emb

        sum_g = jnp.sum(emb, axis=0, keepdims=True)
        t_all = jnp.sum(sum_g * pvec[2:3, :], axis=1, keepdims=True)
        ehi, elo = _split(emb)
        vhi, vlo = _split(pvec[3:4, :])
        t_act_row = (lax.dot_general(vhi, ehi, _C11,
                                     preferred_element_type=jnp.float32) +
                     lax.dot_general(vhi, elo, _C11,
                                     preferred_element_type=jnp.float32) +
                     lax.dot_general(vlo, ehi, _C11,
                                     preferred_element_type=jnp.float32))
        qrow_s[...] = t_act_row + t_all
        cp_q = pltpu.make_async_copy(qrow_s, q_hbm.at[pl.ds(b, 1), :], sem_q)
        cp_q.start()
        cp_q.wait()


@partial(jax.jit, static_argnames=("iters",))
def _graph_embedder(features, weights, adjacency, params, iters=5):
    wsel, wew, wnbw, wnbp, wqall, wqact, wreduc = params
    B, G = features.shape
    E = wsel.shape[0]
    assert iters >= 1 and G % _S == 0

    f = features.astype(jnp.float32)
    w = weights.astype(jnp.float32)
    a = adjacency.astype(jnp.float32)
    wselew = jnp.concatenate([wsel, wew], axis=1).astype(jnp.float32)
    CH = G // _S

    kern = partial(_ge_kernel, iters=iters, G=G, E=E)

    q, emb_flat = pl.pallas_call(
        kern,
        out_shape=(jax.ShapeDtypeStruct((B, G), jnp.float32),
                   jax.ShapeDtypeStruct((B * G, E), jnp.float32)),
        grid_spec=pltpu.PrefetchScalarGridSpec(
            num_scalar_prefetch=0,
            grid=(B, _S),
            in_specs=[
                pl.BlockSpec((B, G), lambda b, s: (0, 0)),
                pl.BlockSpec((1, CH, G), lambda b, s: (b, s, 0)),
                pl.BlockSpec((1, CH, G), lambda b, s: (b, s, 0)),
                pl.BlockSpec((E, 2), lambda b, s: (0, 0)),
                pl.BlockSpec((E, E), lambda b, s: (0, 0)),
                pl.BlockSpec((E, E), lambda b, s: (0, 0)),
                pl.BlockSpec((E, E), lambda b, s: (0, 0)),
                pl.BlockSpec((E, E), lambda b, s: (0, 0)),
                pl.BlockSpec((1, 2 * E), lambda b, s: (0, 0)),
            ],
            out_specs=[
                pl.BlockSpec(memory_space=pltpu.MemorySpace.HBM),
                pl.BlockSpec((G, E), lambda b, s: (b, 0)),
            ],
            scratch_shapes=[
                pltpu.VMEM((G, G), jnp.bfloat16),
                pltpu.VMEM((G, E), jnp.float32),
                pltpu.VMEM((G, 1), jnp.float32),
                pltpu.VMEM((8, E), jnp.float32),
                pltpu.VMEM((1, G), jnp.float32),
                pltpu.SemaphoreType.DMA,
            ],
        ),
        compiler_params=pltpu.CompilerParams(
            dimension_semantics=("parallel", "arbitrary"),
            vmem_limit_bytes=64 * 1024 * 1024),
    )(f, w, a, wselew, wnbw, wnbp, wqall, wqact, wreduc)

    emb = emb_flat.reshape(B, G, E)
    return q, emb


def kernel(features, weights, adjacency, w_selected, w_nbweights_ew,
           w_nbweights, w_nbpriors, w_q_allembed, w_q_action, w_q_reduc):
    params = (w_selected, w_nbweights_ew, w_nbweights, w_nbpriors,
              w_q_allembed, w_q_action, w_q_reduc)
    return _graph_embedder(features, weights, adjacency, params, iters=5)
